# baseline (speedup 1.0000x reference)
import jax
import jax.numpy as jnp
from jax import lax
from jax.experimental import pallas as pl
from jax.experimental.pallas import tpu as pltpu
from jax.experimental.pallas import tpu_sc as plsc

_N = 10000
_E = 320000
_NC = 2
_NS = 16
_NW = _NC * _NS
_K = 80
_CH = _E // (_NW * _K)
_NB = 4
_NI = 8
_RB = 624
_RL = _N - _NS * _RB


def _copy_rows(src_ref, dst_ref, s):
    r0 = s * _RB
    pltpu.sync_copy(src_ref.at[pl.ds(r0, _RB)], dst_ref.at[pl.ds(r0, _RB)])

    @pl.when(s == _NS - 1)
    def _():
        pltpu.sync_copy(src_ref.at[pl.ds(_NS * _RB, _RL)],
                        dst_ref.at[pl.ds(_NS * _RB, _RL)])


def _seg_sum_body(x_hbm, src_hbm, dst_hbm, zeros_hbm, out_hbm,
                  srcb, dstb, rows, acc_sh, isem, dsem, gsem, ssem):
    c = lax.axis_index("c")
    s = lax.axis_index("s")
    w = s * _NC + c
    base_e = w * (_CH * _K)
    _copy_rows(zeros_hbm, acc_sh, s)
    plsc.subcore_barrier()

    def ix_start(j, b):
        off = base_e + j * _K
        pltpu.async_copy(src_hbm.at[pl.ds(off, _K)], srcb.at[b], isem.at[b])
        pltpu.async_copy(dst_hbm.at[pl.ds(off, _K)], dstb.at[b], dsem.at[b])

    def ix_wait_src(j, b):
        off = base_e + j * _K
        pltpu.make_async_copy(src_hbm.at[pl.ds(off, _K)], srcb.at[b],
                              isem.at[b]).wait()

    def ix_wait_dst(j, b):
        off = base_e + j * _K
        pltpu.make_async_copy(dst_hbm.at[pl.ds(off, _K)], dstb.at[b],
                              dsem.at[b]).wait()

    def g_start(ib, b):
        pltpu.async_copy(x_hbm.at[srcb.at[ib]], rows.at[b], gsem.at[b])

    def g_wait(ib, b):
        pltpu.make_async_copy(x_hbm.at[srcb.at[ib]], rows.at[b],
                              gsem.at[b]).wait()

    def s_start(ib, b):
        pltpu.async_copy(rows.at[b], acc_sh.at[dstb.at[ib]], ssem.at[b],
                         add=True)

    def s_wait(ib, b):
        pltpu.make_async_copy(rows.at[b], acc_sh.at[dstb.at[ib]],
                              ssem.at[b]).wait()

    for b in range(_NI):
        ix_start(b, b)
    ix_wait_src(0, 0)
    g_start(0, 0)

    def step(j, u, refill):
        b = u % _NB
        bn = (u + 1) % _NB
        un = (u + 1) % _NI

        @pl.when(j >= _NB - 1)
        def _():
            s_wait((u + 5) % _NI, bn)
        ix_wait_src(j + 1, un)
        g_start(un, bn)
        if refill:
            @pl.when(j >= _NB - 1)
            def _():
                ix_start(j + 5, (u + 5) % _NI)
        g_wait(u, b)
        ix_wait_dst(j, u)
        s_start(u, b)

    def outer(i, carry):
        base = i * _NI
        for u in range(_NI):
            step(base + u, u, True)
        return carry

    nmain = (_CH - 5) // _NI
    lax.fori_loop(0, nmain, outer, 0)
    for j in range(nmain * _NI, _CH - 1):
        step(j, j % _NI, False)
    jt = _CH - 1
    bt = jt % _NB
    g_wait(jt % _NI, bt)
    ix_wait_dst(jt, jt % _NI)
    s_start(jt % _NI, bt)
    for j in range(_CH - _NB, _CH):
        s_wait(j % _NI, j % _NB)

    plsc.subcore_barrier()
    _copy_rows(acc_sh, out_hbm.at[c], s)


def _segment_sum(x, src, dst, zeros, d):
    kern = pl.kernel(
        _seg_sum_body,
        mesh=plsc.VectorSubcoreMesh(core_axis_name="c", subcore_axis_name="s"),
        out_type=jax.ShapeDtypeStruct((_NC, _N, d), jnp.float32),
        scratch_types=[
            pltpu.VMEM((_NI, _K), jnp.int32),
            pltpu.VMEM((_NI, _K), jnp.int32),
            pltpu.VMEM((_NB, _K, d), jnp.float32),
            pltpu.VMEM_SHARED((_N, d), jnp.float32),
            pltpu.SemaphoreType.DMA((_NI,)),
            pltpu.SemaphoreType.DMA((_NI,)),
            pltpu.SemaphoreType.DMA((_NB,)),
            pltpu.SemaphoreType.DMA((_NB,)),
        ],
    )
    return kern(x, src, dst, zeros)


def _bn_math(x_ref, p_ref, w_ref, b_ref, g_ref, be_ref):
    h = x_ref[...] + p_ref[0] + p_ref[1]
    y = jnp.dot(h, w_ref[...], preferred_element_type=jnp.float32) + b_ref[...]
    mu = jnp.mean(y, axis=0, keepdims=True)
    d = y - mu
    var = jnp.mean(d * d, axis=0, keepdims=True)
    return jnp.maximum(g_ref[...] * d * lax.rsqrt(var + 1e-5) + be_ref[...],
                       0.0)


def _dense_bn2_body(x1, p1, w1, b1, g1, be1, x2, p2, w2, b2, g2, be2, o1, o2):
    o1[...] = _bn_math(x1, p1, w1, b1, g1, be1)
    o2[...] = _bn_math(x2, p2, w2, b2, g2, be2)


def _dense_bn2(x1, p1, w1, b1, g1, be1, x2, p2, w2, b2, g2, be2):
    h1, h2 = w1.shape[1], w2.shape[1]
    return pl.pallas_call(
        _dense_bn2_body,
        out_shape=(jax.ShapeDtypeStruct((_N, h1), jnp.float32),
                   jax.ShapeDtypeStruct((_N, h2), jnp.float32)),
    )(x1, p1, w1, b1.reshape(1, h1), g1.reshape(1, h1), be1.reshape(1, h1),
      x2, p2, w2, b2.reshape(1, h2), g2.reshape(1, h2), be2.reshape(1, h2))


def _dense_relu_body(x_ref, p_ref, w_ref, b_ref, o_ref):
    h = x_ref[...] + p_ref[0] + p_ref[1]
    y = jnp.dot(h, w_ref[...], preferred_element_type=jnp.float32) + b_ref[...]
    o_ref[...] = jnp.maximum(y, 0.0)


def _dense_relu(x, p, w, b):
    h = w.shape[1]
    return pl.pallas_call(
        _dense_relu_body,
        out_shape=jax.ShapeDtypeStruct((_N, h), jnp.float32),
    )(x, p, w, b.reshape(1, h))


def kernel(ft, et, fs, es, W_at, b_at, g_at, be_at, W_as, b_as, g_as, be_as,
           W_ex, b_ex, g_ex, be_ex, W_rt, b_rt):
    ft0 = ft[0]
    et0 = et[0]
    src_t = et0[0]
    dst_t = et0[1]
    src_s = es[0]
    dst_s = es[1]
    z128 = jnp.zeros((_N, 128), jnp.float32)
    z64c = jnp.zeros((64,), jnp.float32)
    W_ex_p = jnp.concatenate([W_ex, jnp.zeros((128, 64), jnp.float32)], axis=1)
    b_ex_p = jnp.concatenate([b_ex, z64c])
    g_ex_p = jnp.concatenate([g_ex, z64c])
    be_ex_p = jnp.concatenate([be_ex, z64c])
    W_rt_p = jnp.concatenate([W_rt, jnp.zeros((64, 128), jnp.float32)], axis=0)

    p1 = _segment_sum(ft0, src_t, dst_t, z128, 128)
    p2 = _segment_sum(fs, src_s, dst_s, z128, 128)
    aligned_t, aligned_s = _dense_bn2(ft0, p1, W_at, b_at, g_at, be_at,
                                      fs, p2, W_as, b_as, g_as, be_as)
    p3 = _segment_sum(aligned_t, src_t, dst_t, z128, 128)
    p4 = _segment_sum(aligned_s, src_s, dst_s, z128, 128)
    ht0p, hs = _dense_bn2(aligned_t, p3, W_ex_p, b_ex_p, g_ex_p, be_ex_p,
                          aligned_s, p4, W_ex, b_ex, g_ex, be_ex)
    p5 = _segment_sum(ht0p, src_t, dst_t, z128, 128)
    ft_rec0 = _dense_relu(ht0p, p5, W_rt_p, b_rt)
    return (hs, ht0p[:, :64], ft_rec0, ft0)


# speedup vs baseline: 1.0434x; 1.0434x over previous
import jax
import jax.numpy as jnp
from jax import lax
from jax.experimental import pallas as pl
from jax.experimental.pallas import tpu as pltpu
from jax.experimental.pallas import tpu_sc as plsc

_N = 10000
_E = 320000
_NC = 2
_NS = 16
_NW = _NC * _NS
_K = 80
_CH = _E // (_NW * _K)
_NB = 4
_NI = 8
_RB = 624
_RL = _N - _NS * _RB


def _copy_rows(src_ref, dst_ref, s):
    r0 = s * _RB
    pltpu.sync_copy(src_ref.at[pl.ds(r0, _RB)], dst_ref.at[pl.ds(r0, _RB)])

    @pl.when(s == _NS - 1)
    def _():
        pltpu.sync_copy(src_ref.at[pl.ds(_NS * _RB, _RL)],
                        dst_ref.at[pl.ds(_NS * _RB, _RL)])


def _seg_sum_body(x_hbm, src_hbm, dst_hbm, zeros_hbm, out_hbm,
                  srcb, dstb, rows, acc_sh, isem, dsem, gsem, ssem):
    c = lax.axis_index("c")
    s = lax.axis_index("s")
    w = s * _NC + c
    base_e = w * (_CH * _K)

    def ix_start(j, b):
        off = base_e + j * _K
        pltpu.async_copy(src_hbm.at[pl.ds(off, _K)], srcb.at[b], isem.at[b])
        pltpu.async_copy(dst_hbm.at[pl.ds(off, _K)], dstb.at[b], dsem.at[b])

    def ix_wait_src(j, b):
        off = base_e + j * _K
        pltpu.make_async_copy(src_hbm.at[pl.ds(off, _K)], srcb.at[b],
                              isem.at[b]).wait()

    def ix_wait_dst(j, b):
        off = base_e + j * _K
        pltpu.make_async_copy(dst_hbm.at[pl.ds(off, _K)], dstb.at[b],
                              dsem.at[b]).wait()

    def g_start(ib, b):
        pltpu.async_copy(x_hbm.at[srcb.at[ib]], rows.at[b], gsem.at[b])

    def g_wait(ib, b):
        pltpu.make_async_copy(x_hbm.at[srcb.at[ib]], rows.at[b],
                              gsem.at[b]).wait()

    def s_start(ib, b):
        pltpu.async_copy(rows.at[b], acc_sh.at[dstb.at[ib]], ssem.at[b],
                         add=True)

    def s_wait(ib, b):
        pltpu.make_async_copy(rows.at[b], acc_sh.at[dstb.at[ib]],
                              ssem.at[b]).wait()

    for b in range(_NI):
        ix_start(b, b)
    _copy_rows(zeros_hbm, acc_sh, s)
    plsc.subcore_barrier()
    ix_wait_src(0, 0)
    g_start(0, 0)

    def step(j, u, refill):
        b = u % _NB
        bn = (u + 1) % _NB
        un = (u + 1) % _NI

        @pl.when(j >= _NB - 1)
        def _():
            s_wait((u + 5) % _NI, bn)
        ix_wait_src(j + 1, un)
        g_start(un, bn)
        if refill:
            @pl.when(j >= _NB - 1)
            def _():
                ix_start(j + 5, (u + 5) % _NI)
        g_wait(u, b)
        ix_wait_dst(j, u)
        s_start(u, b)

    def outer(i, carry):
        base = i * _NI
        for u in range(_NI):
            step(base + u, u, True)
        return carry

    nmain = (_CH - 5) // _NI
    lax.fori_loop(0, nmain, outer, 0)
    for j in range(nmain * _NI, _CH - 1):
        step(j, j % _NI, False)
    jt = _CH - 1
    bt = jt % _NB
    g_wait(jt % _NI, bt)
    ix_wait_dst(jt, jt % _NI)
    s_start(jt % _NI, bt)
    for j in range(_CH - _NB, _CH):
        s_wait(j % _NI, j % _NB)

    plsc.subcore_barrier()
    _copy_rows(acc_sh, out_hbm.at[c], s)


def _segment_sum(x, src, dst, zeros, d):
    kern = pl.kernel(
        _seg_sum_body,
        mesh=plsc.VectorSubcoreMesh(core_axis_name="c", subcore_axis_name="s"),
        out_type=jax.ShapeDtypeStruct((_NC, _N, d), jnp.float32),
        scratch_types=[
            pltpu.VMEM((_NI, _K), jnp.int32),
            pltpu.VMEM((_NI, _K), jnp.int32),
            pltpu.VMEM((_NB, _K, d), jnp.float32),
            pltpu.VMEM_SHARED((_N, d), jnp.float32),
            pltpu.SemaphoreType.DMA((_NI,)),
            pltpu.SemaphoreType.DMA((_NI,)),
            pltpu.SemaphoreType.DMA((_NB,)),
            pltpu.SemaphoreType.DMA((_NB,)),
        ],
    )
    return kern(x, src, dst, zeros)


def _bn_math(x_ref, p_ref, w_ref, b_ref, g_ref, be_ref):
    h = x_ref[...] + p_ref[0] + p_ref[1]
    y = jnp.dot(h, w_ref[...], preferred_element_type=jnp.float32) + b_ref[...]
    mu = jnp.mean(y, axis=0, keepdims=True)
    d = y - mu
    var = jnp.mean(d * d, axis=0, keepdims=True)
    return jnp.maximum(g_ref[...] * d * lax.rsqrt(var + 1e-5) + be_ref[...],
                       0.0)


def _dense_bn_body(x_ref, p_ref, w_ref, b_ref, g_ref, be_ref, o_ref):
    o_ref[...] = _bn_math(x_ref, p_ref, w_ref, b_ref, g_ref, be_ref)


def _dense_bn(x, p, w, b, g, be):
    h = w.shape[1]
    return pl.pallas_call(
        _dense_bn_body,
        out_shape=jax.ShapeDtypeStruct((_N, h), jnp.float32),
    )(x, p, w, b.reshape(1, h), g.reshape(1, h), be.reshape(1, h))


def _dense_relu_body(x_ref, p_ref, w_ref, b_ref, o_ref):
    h = x_ref[...] + p_ref[0] + p_ref[1]
    y = jnp.dot(h, w_ref[...], preferred_element_type=jnp.float32) + b_ref[...]
    o_ref[...] = jnp.maximum(y, 0.0)


def _dense_relu(x, p, w, b):
    h = w.shape[1]
    return pl.pallas_call(
        _dense_relu_body,
        out_shape=jax.ShapeDtypeStruct((_N, h), jnp.float32),
    )(x, p, w, b.reshape(1, h))


def kernel(ft, et, fs, es, W_at, b_at, g_at, be_at, W_as, b_as, g_as, be_as,
           W_ex, b_ex, g_ex, be_ex, W_rt, b_rt):
    ft0 = ft[0]
    et0 = et[0]
    src_t = et0[0]
    dst_t = et0[1]
    src_s = es[0]
    dst_s = es[1]
    z128 = jnp.zeros((_N, 128), jnp.float32)
    z64c = jnp.zeros((64,), jnp.float32)
    W_ex_p = jnp.concatenate([W_ex, jnp.zeros((128, 64), jnp.float32)], axis=1)
    b_ex_p = jnp.concatenate([b_ex, z64c])
    g_ex_p = jnp.concatenate([g_ex, z64c])
    be_ex_p = jnp.concatenate([be_ex, z64c])
    W_rt_p = jnp.concatenate([W_rt, jnp.zeros((64, 128), jnp.float32)], axis=0)

    p = _segment_sum(ft0, src_t, dst_t, z128, 128)
    aligned_t = _dense_bn(ft0, p, W_at, b_at, g_at, be_at)
    p = _segment_sum(fs, src_s, dst_s, z128, 128)
    aligned_s = _dense_bn(fs, p, W_as, b_as, g_as, be_as)
    p = _segment_sum(aligned_t, src_t, dst_t, z128, 128)
    ht0p = _dense_bn(aligned_t, p, W_ex_p, b_ex_p, g_ex_p, be_ex_p)
    p = _segment_sum(aligned_s, src_s, dst_s, z128, 128)
    hs = _dense_bn(aligned_s, p, W_ex, b_ex, g_ex, be_ex)
    p = _segment_sum(ht0p, src_t, dst_t, z128, 128)
    ft_rec0 = _dense_relu(ht0p, p, W_rt_p, b_rt)
    return (hs, ht0p[:, :64], ft_rec0, ft0)
